# Initial kernel scaffold; baseline (speedup 1.0000x reference)
import jax
import jax.numpy as jnp
from jax.experimental import pallas as pl


def kernel(hidden_states, position_ids, q_w, k_w, v_w, o_w):
    raise NotImplementedError("write your pallas kernel here")


# speedup vs baseline: 892.4539x; 892.4539x over previous
import functools
import math

import jax
import jax.numpy as jnp
from jax.experimental import pallas as pl

HID = 2048
NH = 16
HD = HID // NH
S = 2048
BQ = 256
SCALE = 1.0 / math.sqrt(HD)
NEG = -1e30
THETA = 10000.0


def _proj_body(x_ref, w_ref, cos_ref, sin_ref, o_ref, *, rope):
    y = jax.lax.dot_general(
        x_ref[...], w_ref[...], (((1,), (1,)), ((), ())),
        preferred_element_type=jnp.float32)
    if rope:
        c = cos_ref[...]
        s = sin_ref[...]
        x1 = y[:, :HD // 2]
        x2 = y[:, HD // 2:]
        rot = jnp.concatenate([-x2, x1], axis=1)
        y = y * c + rot * s
    o_ref[...] = y


def _attn_body(q_ref, k_ref, v_ref, o_ref):
    j = pl.program_id(1)
    q = q_ref[...]
    s = jax.lax.dot_general(
        q, k_ref[...], (((1,), (1,)), ((), ())),
        preferred_element_type=jnp.float32) * SCALE
    row = j * BQ + jax.lax.broadcasted_iota(jnp.int32, (BQ, S), 0)
    col = jax.lax.broadcasted_iota(jnp.int32, (BQ, S), 1)
    s = jnp.where(col <= row, s, NEG)
    m = jnp.max(s, axis=1, keepdims=True)
    p = jnp.exp(s - m)
    l = jnp.sum(p, axis=1, keepdims=True)
    o = jax.lax.dot_general(
        p, v_ref[...], (((1,), (0,)), ((), ())),
        preferred_element_type=jnp.float32)
    o_ref[...] = o / l


def _oproj_body(a_ref, w_ref, o_ref):
    o_ref[...] = jax.lax.dot_general(
        a_ref[...], w_ref[...], (((1,), (1,)), ((), ())),
        preferred_element_type=jnp.float32)


def kernel(hidden_states, position_ids, q_w, k_w, v_w, o_w):
    x = hidden_states[0]
    pos = position_ids[0].astype(jnp.float32)
    inv_freq = 1.0 / (THETA ** (jnp.arange(0, HD, 2, dtype=jnp.float32) / HD))
    freqs = pos[:, None] * inv_freq[None, :]
    emb = jnp.concatenate([freqs, freqs], axis=-1)
    cos = jnp.cos(emb)
    sin = jnp.sin(emb)

    def proj(w, rope):
        return pl.pallas_call(
            functools.partial(_proj_body, rope=rope),
            grid=(NH,),
            in_specs=[
                pl.BlockSpec((S, HID), lambda h: (0, 0)),
                pl.BlockSpec((HD, HID), lambda h: (h, 0)),
                pl.BlockSpec((S, HD), lambda h: (0, 0)),
                pl.BlockSpec((S, HD), lambda h: (0, 0)),
            ],
            out_specs=pl.BlockSpec((S, HD), lambda h: (0, h)),
            out_shape=jax.ShapeDtypeStruct((S, NH * HD), jnp.float32),
        )(x, w, cos, sin)

    q = proj(q_w, True)
    k = proj(k_w, True)
    v = proj(v_w, False)

    attn = pl.pallas_call(
        _attn_body,
        grid=(NH, S // BQ),
        in_specs=[
            pl.BlockSpec((BQ, HD), lambda h, j: (j, h)),
            pl.BlockSpec((S, HD), lambda h, j: (0, h)),
            pl.BlockSpec((S, HD), lambda h, j: (0, h)),
        ],
        out_specs=pl.BlockSpec((BQ, HD), lambda h, j: (j, h)),
        out_shape=jax.ShapeDtypeStruct((S, NH * HD), jnp.float32),
    )(q, k, v)

    out = pl.pallas_call(
        _oproj_body,
        grid=(HID // HD,),
        in_specs=[
            pl.BlockSpec((S, NH * HD), lambda h: (0, 0)),
            pl.BlockSpec((HD, NH * HD), lambda h: (h, 0)),
        ],
        out_specs=pl.BlockSpec((S, HD), lambda h: (0, h)),
        out_shape=jax.ShapeDtypeStruct((S, HID), jnp.float32),
    )(attn, o_w)

    return out[None]


# speedup vs baseline: 927.6012x; 1.0394x over previous
import functools
import math

import jax
import jax.numpy as jnp
from jax.experimental import pallas as pl

HID = 2048
NH = 16
HD = HID // NH
S = 2048
BQ = 256
SCALE = 1.0 / math.sqrt(HD)
NEG = -1e30
THETA = 10000.0


def _proj_body(x_ref, w_ref, cos_ref, sin_ref, o_ref, *, rope):
    y = jax.lax.dot_general(
        x_ref[...], w_ref[...], (((1,), (1,)), ((), ())),
        preferred_element_type=jnp.float32)
    if rope:
        c = cos_ref[...]
        s = sin_ref[...]
        x1 = y[:, :HD // 2]
        x2 = y[:, HD // 2:]
        rot = jnp.concatenate([-x2, x1], axis=1)
        y = y * c + rot * s
    o_ref[...] = y.astype(jnp.bfloat16)


def _attn_body(q_ref, k_ref, v_ref, o_ref):
    j = pl.program_id(1)
    q = q_ref[...]
    s = jax.lax.dot_general(
        q, k_ref[...], (((1,), (1,)), ((), ())),
        preferred_element_type=jnp.float32) * SCALE
    row = j * BQ + jax.lax.broadcasted_iota(jnp.int32, (BQ, S), 0)
    col = jax.lax.broadcasted_iota(jnp.int32, (BQ, S), 1)
    s = jnp.where(col <= row, s, NEG)
    m = jnp.max(s, axis=1, keepdims=True)
    p = jnp.exp(s - m)
    l = jnp.sum(p, axis=1, keepdims=True)
    o = jax.lax.dot_general(
        p.astype(jnp.bfloat16), v_ref[...], (((1,), (0,)), ((), ())),
        preferred_element_type=jnp.float32)
    o_ref[...] = (o / l).astype(jnp.bfloat16)


def _oproj_body(a_ref, w_ref, o_ref):
    o_ref[...] = jax.lax.dot_general(
        a_ref[...], w_ref[...], (((1,), (1,)), ((), ())),
        preferred_element_type=jnp.float32)


def kernel(hidden_states, position_ids, q_w, k_w, v_w, o_w):
    x = hidden_states[0].astype(jnp.bfloat16)
    q_w = q_w.astype(jnp.bfloat16)
    k_w = k_w.astype(jnp.bfloat16)
    v_w = v_w.astype(jnp.bfloat16)
    o_w = o_w.astype(jnp.bfloat16)
    pos = position_ids[0].astype(jnp.float32)
    inv_freq = 1.0 / (THETA ** (jnp.arange(0, HD, 2, dtype=jnp.float32) / HD))
    freqs = pos[:, None] * inv_freq[None, :]
    emb = jnp.concatenate([freqs, freqs], axis=-1)
    cos = jnp.cos(emb)
    sin = jnp.sin(emb)

    def proj(w, rope):
        return pl.pallas_call(
            functools.partial(_proj_body, rope=rope),
            grid=(NH,),
            in_specs=[
                pl.BlockSpec((S, HID), lambda h: (0, 0)),
                pl.BlockSpec((HD, HID), lambda h: (h, 0)),
                pl.BlockSpec((S, HD), lambda h: (0, 0)),
                pl.BlockSpec((S, HD), lambda h: (0, 0)),
            ],
            out_specs=pl.BlockSpec((S, HD), lambda h: (0, h)),
            out_shape=jax.ShapeDtypeStruct((S, NH * HD), jnp.bfloat16),
        )(x, w, cos, sin)

    q = proj(q_w, True)
    k = proj(k_w, True)
    v = proj(v_w, False)

    attn = pl.pallas_call(
        _attn_body,
        grid=(NH, S // BQ),
        in_specs=[
            pl.BlockSpec((BQ, HD), lambda h, j: (j, h)),
            pl.BlockSpec((S, HD), lambda h, j: (0, h)),
            pl.BlockSpec((S, HD), lambda h, j: (0, h)),
        ],
        out_specs=pl.BlockSpec((BQ, HD), lambda h, j: (j, h)),
        out_shape=jax.ShapeDtypeStruct((S, NH * HD), jnp.bfloat16),
    )(q, k, v)

    out = pl.pallas_call(
        _oproj_body,
        grid=(HID // HD,),
        in_specs=[
            pl.BlockSpec((S, NH * HD), lambda h: (0, 0)),
            pl.BlockSpec((HD, NH * HD), lambda h: (h, 0)),
        ],
        out_specs=pl.BlockSpec((S, HD), lambda h: (0, h)),
        out_shape=jax.ShapeDtypeStruct((S, HID), jnp.float32),
    )(attn, o_w)

    return out[None]
